# baseline (speedup 1.0000x reference)
import functools

import jax
import jax.numpy as jnp
from jax import lax
from jax.experimental import pallas as pl
from jax.experimental.pallas import tpu as pltpu
from jax.experimental.pallas import tpu_sc as plsc

N = 500000
D = 64
R = 768
NCHUNK = 651
TAIL = N - R * NCHUNK
NC = 2
NS = 16
NW = NC * NS
NB = 2
TPW = 22


def _body(numbers_hbm, mapper_hbm, eye_hbm, out_hbm,
          map_v, eye_v, z0, z1, zt_v, c0, c1, buf0, buf1, tail_v,
          sem0, sem1, zsem0, zsem1):
    bufs = (buf0, buf1)
    zs = (z0, z1)
    csaves = (c0, c1)
    sems = (sem0, sem1)
    zsems = (zsem0, zsem1)
    wid = lax.axis_index("s") * NC + lax.axis_index("c")
    lane = lax.broadcasted_iota(jnp.int32, (16,), 0)
    zeros16 = jnp.zeros((16,), jnp.float32)

    for b in range(NB):
        pltpu.async_copy(
            numbers_hbm.at[pl.ds((wid + b * NW) * R, R)], zs[b], zsems[b])

    pltpu.sync_copy(mapper_hbm, map_v)
    pltpu.sync_copy(eye_hbm, eye_v)

    def zero_row(c, _):
        for buf in bufs:
            for k in range(R // 16):
                buf[c, pl.ds(k * 16, 16)] = zeros16
        for k in range(TAIL // 16):
            tail_v[c, pl.ds(k * 16, 16)] = zeros16
        return 0

    lax.fori_loop(0, D, zero_row, 0)

    def zero_csave(k, _):
        izeros = jnp.zeros((16,), jnp.int32)
        for c_v in csaves:
            c_v[pl.ds(k * 16, 16)] = izeros
        return 0

    lax.fori_loop(0, R // 16, zero_csave, 0)

    def round_body(tt, _):
        for b in range(NB):
            chunk = wid + (NB * tt + b) * NW

            @pl.when(chunk < NCHUNK)
            def _(b=b, chunk=chunk):
                buf, z_v, c_v = bufs[b], zs[b], csaves[b]
                sem, zsem = sems[b], zsems[b]
                base = chunk * R
                pltpu.make_async_copy(
                    numbers_hbm.at[pl.ds(0, R)], z_v, zsem).wait()

                @pl.when(tt > 0)
                def _():
                    pltpu.make_async_copy(
                        buf, out_hbm.at[:, pl.ds(0, R)], sem).wait()

                @plsc.parallel_loop(0, R // 16, unroll=4)
                def _(j):
                    col = j * 16 + lane
                    cold = c_v[pl.ds(j * 16, 16)]
                    plsc.store_scatter(buf, [cold, col], zeros16)

                @plsc.parallel_loop(0, R // 16, unroll=4)
                def _(j):
                    col = j * 16 + lane
                    z = z_v[pl.ds(j * 16, 16)]
                    c = plsc.load_gather(map_v, [z])
                    val = plsc.load_gather(eye_v, [c, c])
                    plsc.store_scatter(buf, [c, col], val)
                    c_v[pl.ds(j * 16, 16)] = c

                nxt = chunk + NB * NW

                @pl.when(nxt < NCHUNK)
                def _():
                    pltpu.async_copy(
                        numbers_hbm.at[pl.ds(nxt * R, R)], z_v, zsem)

                pltpu.async_copy(buf, out_hbm.at[:, pl.ds(base, R)], sem)

        return 0

    lax.fori_loop(0, TPW // NB, round_body, 0)

    @pl.when(wid == NW - 1)
    def _():
        pltpu.sync_copy(numbers_hbm.at[pl.ds(R * NCHUNK, TAIL)], zt_v)
        for j in range(TAIL // 16):
            z = zt_v[pl.ds(j * 16, 16)]
            c = plsc.load_gather(map_v, [z])
            col = j * 16 + lane
            val = plsc.load_gather(eye_v, [c, c])
            plsc.store_scatter(tail_v, [c, col], val)
        pltpu.sync_copy(tail_v, out_hbm.at[:, pl.ds(R * NCHUNK, TAIL)])

    for b in range(NB):
        pltpu.make_async_copy(bufs[b], out_hbm.at[:, pl.ds(0, R)], sems[b]).wait()


@jax.jit
def kernel(numbers, mapper, eye_matrix):
    run = functools.partial(
        pl.kernel,
        out_type=jax.ShapeDtypeStruct((D, N), jnp.float32),
        mesh=plsc.VectorSubcoreMesh(core_axis_name="c", subcore_axis_name="s"),
        compiler_params=pltpu.CompilerParams(needs_layout_passes=False),
        scratch_types=[
            pltpu.VMEM((65,), jnp.int32),
            pltpu.VMEM((D, D), jnp.float32),
            pltpu.VMEM((R,), jnp.int32),
            pltpu.VMEM((R,), jnp.int32),
            pltpu.VMEM((TAIL,), jnp.int32),
            pltpu.VMEM((R,), jnp.int32),
            pltpu.VMEM((R,), jnp.int32),
            pltpu.VMEM((D, R), jnp.float32),
            pltpu.VMEM((D, R), jnp.float32),
            pltpu.VMEM((D, TAIL), jnp.float32),
            pltpu.SemaphoreType.DMA,
            pltpu.SemaphoreType.DMA,
            pltpu.SemaphoreType.DMA,
            pltpu.SemaphoreType.DMA,
        ],
    )(_body)
    return run(numbers, mapper, eye_matrix).T


# speedup vs baseline: 1.0309x; 1.0309x over previous
import functools

import jax
import jax.numpy as jnp
from jax import lax
from jax.experimental import pallas as pl
from jax.experimental.pallas import tpu as pltpu
from jax.experimental.pallas import tpu_sc as plsc

N = 500000
D = 64
R = 768
NCHUNK = 651
TAIL = N - R * NCHUNK
NC = 2
NS = 16
NW = NC * NS
NB = 2
TPW = 22


def _body(numbers_hbm, mapper_hbm, eye_hbm, out_hbm,
          map_v, eye_v, z0, z1, zt_v, c0, c1, buf0, buf1, tail_v,
          sem0, sem1, zsem0, zsem1):
    bufs = (buf0, buf1)
    zs = (z0, z1)
    csaves = (c0, c1)
    sems = (sem0, sem1)
    zsems = (zsem0, zsem1)
    wid = lax.axis_index("s") * NC + lax.axis_index("c")
    lane = lax.broadcasted_iota(jnp.int32, (16,), 0)
    zeros16 = jnp.zeros((16,), jnp.float32)

    for b in range(NB):
        pltpu.async_copy(
            numbers_hbm.at[pl.ds((wid + b * NW) * R, R)], zs[b], zsems[b])

    pltpu.sync_copy(mapper_hbm, map_v)
    pltpu.sync_copy(eye_hbm, eye_v)

    def zero_buf(buf):
        def zero_row(c, _):
            for k in range(R // 16):
                buf[c, pl.ds(k * 16, 16)] = zeros16
            return 0

        lax.fori_loop(0, D, zero_row, 0)

    def process(b, chunk, clear_old):
        buf, z_v, c_v = bufs[b], zs[b], csaves[b]
        sem, zsem = sems[b], zsems[b]
        base = chunk * R
        pltpu.make_async_copy(numbers_hbm.at[pl.ds(0, R)], z_v, zsem).wait()

        if clear_old:
            pltpu.make_async_copy(buf, out_hbm.at[:, pl.ds(0, R)], sem).wait()

            @plsc.parallel_loop(0, R // 16, unroll=4)
            def _(j):
                col = j * 16 + lane
                cold = c_v[pl.ds(j * 16, 16)]
                plsc.store_scatter(buf, [cold, col], zeros16)

        @plsc.parallel_loop(0, R // 16, unroll=4)
        def _(j):
            col = j * 16 + lane
            z = z_v[pl.ds(j * 16, 16)]
            c = plsc.load_gather(map_v, [z])
            val = plsc.load_gather(eye_v, [c, c])
            plsc.store_scatter(buf, [c, col], val)
            c_v[pl.ds(j * 16, 16)] = c

        nxt = chunk + NB * NW

        @pl.when(nxt < NCHUNK)
        def _():
            pltpu.async_copy(numbers_hbm.at[pl.ds(nxt * R, R)], z_v, zsem)

        pltpu.async_copy(buf, out_hbm.at[:, pl.ds(base, R)], sem)

    zero_buf(buf0)
    process(0, wid, clear_old=False)
    zero_buf(buf1)

    def zero_tail_row(c, _):
        for k in range(TAIL // 16):
            tail_v[c, pl.ds(k * 16, 16)] = zeros16
        return 0

    lax.fori_loop(0, D, zero_tail_row, 0)
    process(1, wid + NW, clear_old=False)

    def round_body(tt, _):
        for b in range(NB):
            chunk = wid + (NB * tt + b) * NW

            @pl.when(chunk < NCHUNK)
            def _(b=b, chunk=chunk):
                process(b, chunk, clear_old=True)

        return 0

    lax.fori_loop(1, TPW // NB, round_body, 0)

    @pl.when(wid == NW - 1)
    def _():
        pltpu.sync_copy(numbers_hbm.at[pl.ds(R * NCHUNK, TAIL)], zt_v)
        for j in range(TAIL // 16):
            z = zt_v[pl.ds(j * 16, 16)]
            c = plsc.load_gather(map_v, [z])
            col = j * 16 + lane
            val = plsc.load_gather(eye_v, [c, c])
            plsc.store_scatter(tail_v, [c, col], val)
        pltpu.sync_copy(tail_v, out_hbm.at[:, pl.ds(R * NCHUNK, TAIL)])

    for b in range(NB):
        pltpu.make_async_copy(bufs[b], out_hbm.at[:, pl.ds(0, R)], sems[b]).wait()


@jax.jit
def kernel(numbers, mapper, eye_matrix):
    run = functools.partial(
        pl.kernel,
        out_type=jax.ShapeDtypeStruct((D, N), jnp.float32),
        mesh=plsc.VectorSubcoreMesh(core_axis_name="c", subcore_axis_name="s"),
        compiler_params=pltpu.CompilerParams(needs_layout_passes=False),
        scratch_types=[
            pltpu.VMEM((65,), jnp.int32),
            pltpu.VMEM((D, D), jnp.float32),
            pltpu.VMEM((R,), jnp.int32),
            pltpu.VMEM((R,), jnp.int32),
            pltpu.VMEM((TAIL,), jnp.int32),
            pltpu.VMEM((R,), jnp.int32),
            pltpu.VMEM((R,), jnp.int32),
            pltpu.VMEM((D, R), jnp.float32),
            pltpu.VMEM((D, R), jnp.float32),
            pltpu.VMEM((D, TAIL), jnp.float32),
            pltpu.SemaphoreType.DMA,
            pltpu.SemaphoreType.DMA,
            pltpu.SemaphoreType.DMA,
            pltpu.SemaphoreType.DMA,
        ],
    )(_body)
    return run(numbers, mapper, eye_matrix).T
